# Initial kernel scaffold; baseline (speedup 1.0000x reference)
import functools

import jax
import jax.numpy as jnp
from jax import lax
from jax.experimental import pallas as pl
from jax.experimental.pallas import tpu as pltpu
from jax.experimental.pallas import tpu_sc as plsc

N_ATOMS = 1600000
N_SEG = 10000
OUT_U = 16

N_CORES = 2
N_SUBCORES = 16
NW = N_CORES * N_SUBCORES
ATOMS_PER_W = N_ATOMS // NW
BLOCK = 2000
NBLK = ATOMS_PER_W // BLOCK
SUB = 80
NSUB = BLOCK // SUB
SEG_PER_TILE = N_SEG // N_SUBCORES

_mesh = plsc.VectorSubcoreMesh(core_axis_name="c", subcore_axis_name="s")


@functools.partial(
    pl.kernel,
    out_type=jax.ShapeDtypeStruct((N_CORES, N_SEG, OUT_U), jnp.float32),
    mesh=_mesh,
    scratch_types=[
        pltpu.VMEM_SHARED((N_SEG, OUT_U), jnp.float32),
        pltpu.VMEM((SEG_PER_TILE, OUT_U), jnp.float32),
        pltpu.VMEM((NSUB, SUB), jnp.int32),
        pltpu.VMEM((BLOCK, OUT_U), jnp.float32),
    ],
)
def _sc_segsum(ids_hbm, rows_hbm, out_hbm, acc_sh, zero_v, ids_v, rows_v):
    c = lax.axis_index("c")
    s = lax.axis_index("s")
    w = c * N_SUBCORES + s

    def _zero_row(i, carry):
        zero_v[i, :] = jnp.zeros((OUT_U,), jnp.float32)
        return carry

    lax.fori_loop(0, SEG_PER_TILE, _zero_row, 0)
    pltpu.sync_copy(zero_v, acc_sh.at[pl.ds(s * SEG_PER_TILE, SEG_PER_TILE)])
    plsc.subcore_barrier()

    def _block(b, carry):
        base = w * ATOMS_PER_W + b * BLOCK
        pltpu.sync_copy(rows_hbm.at[pl.ds(base, BLOCK)], rows_v)
        pltpu.sync_copy(ids_hbm.at[w, b], ids_v)

        def _sub(j, inner):
            pltpu.sync_copy(
                rows_v.at[pl.ds(j * SUB, SUB)],
                acc_sh.at[ids_v.at[j]],
                add=True,
            )
            return inner

        lax.fori_loop(0, NSUB, _sub, 0)
        return carry

    lax.fori_loop(0, NBLK, _block, 0)
    plsc.subcore_barrier()

    pltpu.sync_copy(
        acc_sh.at[pl.ds(s * SEG_PER_TILE, SEG_PER_TILE)],
        out_hbm.at[c, pl.ds(s * SEG_PER_TILE, SEG_PER_TILE)],
    )


def _tc_combine(p_ref, o_ref):
    o_ref[...] = p_ref[0] + p_ref[1]


_combine = pl.pallas_call(
    _tc_combine,
    out_shape=jax.ShapeDtypeStruct((N_SEG, OUT_U), jnp.float32),
)


def kernel(ind_1, output):
    batch = ind_1[:, 0] if ind_1.ndim == 2 else ind_1
    ids = batch.astype(jnp.int32).reshape(NW, NBLK, NSUB, SUB)
    partials = _sc_segsum(ids, output)
    return _combine(partials)


# speedup vs baseline: 6.2706x; 6.2706x over previous
import functools

import jax
import jax.numpy as jnp
from jax import lax
from jax.experimental import pallas as pl
from jax.experimental.pallas import tpu as pltpu
from jax.experimental.pallas import tpu_sc as plsc

N_ATOMS = 1600000
N_SEG = 10000
OUT_U = 16

N_CORES = 2
N_SUBCORES = 16
NW = N_CORES * N_SUBCORES
ATOMS_PER_W = N_ATOMS // NW
BLOCK = 2000
NBLK = ATOMS_PER_W // BLOCK
SUB = 80
NSUB = BLOCK // SUB
SEG_PER_TILE = 632
N_SEG_PAD = SEG_PER_TILE * N_SUBCORES

_mesh = plsc.VectorSubcoreMesh(core_axis_name="c", subcore_axis_name="s")


@functools.partial(
    pl.kernel,
    out_type=jax.ShapeDtypeStruct((N_CORES, N_SEG_PAD, OUT_U), jnp.float32),
    mesh=_mesh,
    scratch_types=[
        pltpu.VMEM_SHARED((N_SEG_PAD, OUT_U), jnp.float32),
        pltpu.VMEM((SEG_PER_TILE, OUT_U), jnp.float32),
        pltpu.VMEM((NSUB, SUB), jnp.int32),
        pltpu.VMEM((BLOCK, OUT_U), jnp.float32),
    ],
    compiler_params=pltpu.CompilerParams(use_tc_tiling_on_sc=False),
)
def _sc_segsum(ids_hbm, rows_hbm, out_hbm, acc_sh, zero_v, ids_v, rows_v):
    c = lax.axis_index("c")
    s = lax.axis_index("s")
    w = c * N_SUBCORES + s

    def _zero_row(i, carry):
        zero_v[i, :] = jnp.zeros((OUT_U,), jnp.float32)
        return carry

    lax.fori_loop(0, SEG_PER_TILE, _zero_row, 0)
    pltpu.sync_copy(zero_v, acc_sh.at[pl.ds(s * SEG_PER_TILE, SEG_PER_TILE)])
    plsc.subcore_barrier()

    def _block(b, carry):
        base = w * ATOMS_PER_W + b * BLOCK
        pltpu.sync_copy(rows_hbm.at[pl.ds(base, BLOCK)], rows_v)
        pltpu.sync_copy(ids_hbm.at[w, b], ids_v)

        def _sub(j, inner):
            pltpu.sync_copy(
                rows_v.at[pl.ds(j * SUB, SUB)],
                acc_sh.at[ids_v.at[j]],
                add=True,
            )
            return inner

        lax.fori_loop(0, NSUB, _sub, 0)
        return carry

    lax.fori_loop(0, NBLK, _block, 0)
    plsc.subcore_barrier()

    pltpu.sync_copy(
        acc_sh.at[pl.ds(s * SEG_PER_TILE, SEG_PER_TILE)],
        out_hbm.at[c, pl.ds(s * SEG_PER_TILE, SEG_PER_TILE)],
    )


def _tc_combine(p_ref, o_ref):
    o_ref[...] = p_ref[0, :N_SEG] + p_ref[1, :N_SEG]


_combine = pl.pallas_call(
    _tc_combine,
    out_shape=jax.ShapeDtypeStruct((N_SEG, OUT_U), jnp.float32),
)


def kernel(ind_1, output):
    batch = ind_1[:, 0] if ind_1.ndim == 2 else ind_1
    ids = batch.astype(jnp.int32).reshape(NW, NBLK, NSUB, SUB)
    partials = _sc_segsum(ids, output)
    return _combine(partials)


# speedup vs baseline: 6.3187x; 1.0077x over previous
import functools

import jax
import jax.numpy as jnp
from jax import lax
from jax.experimental import pallas as pl
from jax.experimental.pallas import tpu as pltpu
from jax.experimental.pallas import tpu_sc as plsc

N_ATOMS = 1600000
N_SEG = 10000
OUT_U = 16

N_CORES = 2
N_SUBCORES = 16
NW = N_CORES * N_SUBCORES
ATOMS_PER_W = N_ATOMS // NW
BLOCK = 2000
NBLK = ATOMS_PER_W // BLOCK
SUB = 80
NSUB = BLOCK // SUB
SEG_PER_TILE = 632
N_SEG_PAD = SEG_PER_TILE * N_SUBCORES

_mesh = plsc.VectorSubcoreMesh(core_axis_name="c", subcore_axis_name="s")


@functools.partial(
    pl.kernel,
    out_type=jax.ShapeDtypeStruct((N_CORES, N_SEG_PAD, OUT_U), jnp.float32),
    mesh=_mesh,
    scratch_types=[
        pltpu.VMEM_SHARED((N_SEG_PAD, OUT_U), jnp.float32),
        pltpu.VMEM((SEG_PER_TILE, OUT_U), jnp.float32),
        pltpu.VMEM((BLOCK,), jnp.int32),
        pltpu.VMEM((BLOCK, OUT_U), jnp.float32),
    ],
    compiler_params=pltpu.CompilerParams(use_tc_tiling_on_sc=False),
)
def _sc_segsum(ids_hbm, rows_hbm, out_hbm, acc_sh, zero_v, ids_v, rows_v):
    c = lax.axis_index("c")
    s = lax.axis_index("s")
    w = c * N_SUBCORES + s

    def _zero_row(i, carry):
        zero_v[i, :] = jnp.zeros((OUT_U,), jnp.float32)
        return carry

    lax.fori_loop(0, SEG_PER_TILE, _zero_row, 0)
    pltpu.sync_copy(zero_v, acc_sh.at[pl.ds(s * SEG_PER_TILE, SEG_PER_TILE)])
    plsc.subcore_barrier()

    def _block(b, carry):
        base = w * ATOMS_PER_W + b * BLOCK
        pltpu.sync_copy(rows_hbm.at[pl.ds(base, BLOCK)], rows_v)
        pltpu.sync_copy(ids_hbm.at[pl.ds(base, BLOCK)], ids_v)

        def _sub(j, inner):
            pltpu.sync_copy(
                rows_v.at[pl.ds(j * SUB, SUB)],
                acc_sh.at[ids_v.at[pl.ds(j * SUB, SUB)]],
                add=True,
            )
            return inner

        lax.fori_loop(0, NSUB, _sub, 0)
        return carry

    lax.fori_loop(0, NBLK, _block, 0)
    plsc.subcore_barrier()

    pltpu.sync_copy(
        acc_sh.at[pl.ds(s * SEG_PER_TILE, SEG_PER_TILE)],
        out_hbm.at[c, pl.ds(s * SEG_PER_TILE, SEG_PER_TILE)],
    )


CMB_ROWS = 320
CMB_TAIL = N_SEG - (NW - 1) * CMB_ROWS


@functools.partial(
    pl.kernel,
    out_type=jax.ShapeDtypeStruct((N_SEG, OUT_U), jnp.float32),
    mesh=_mesh,
    scratch_types=[
        pltpu.VMEM((CMB_ROWS, OUT_U), jnp.float32),
        pltpu.VMEM((CMB_ROWS, OUT_U), jnp.float32),
    ],
    compiler_params=pltpu.CompilerParams(use_tc_tiling_on_sc=False),
)
def _sc_combine(p_hbm, out_hbm, a_v, b_v):
    c = lax.axis_index("c")
    s = lax.axis_index("s")
    w = c * N_SUBCORES + s
    base = w * CMB_ROWS

    def _do(nrows):
        pltpu.sync_copy(p_hbm.at[0, pl.ds(base, nrows)], a_v.at[pl.ds(0, nrows)])
        pltpu.sync_copy(p_hbm.at[1, pl.ds(base, nrows)], b_v.at[pl.ds(0, nrows)])

        def _add(i, carry):
            a_v[i, :] = a_v[i, :] + b_v[i, :]
            return carry

        lax.fori_loop(0, nrows, _add, 0)
        pltpu.sync_copy(a_v.at[pl.ds(0, nrows)], out_hbm.at[pl.ds(base, nrows)])

    @pl.when(w < NW - 1)
    def _full():
        _do(CMB_ROWS)

    @pl.when(w == NW - 1)
    def _tail():
        _do(CMB_TAIL)


def kernel(ind_1, output):
    batch = ind_1[:, 0] if ind_1.ndim == 2 else ind_1
    ids = batch.astype(jnp.int32)
    partials = _sc_segsum(ids, output)
    return _sc_combine(partials)
